# baseline (device time: 99905 ns/iter reference)
import jax
import jax.numpy as jnp
from jax import lax
from jax.experimental import pallas as pl
from jax.experimental.pallas import tpu as pltpu

N_DEV = 4


def kernel(A, B):
    m, k = A.shape
    _, n = B.shape
    ch = m // N_DEV
    n_steps = 2 * (N_DEV - 1)

    def body(a_ref, b_ref, out_ref, bbf_ref, comm_ref, send_sems, recv_sems):
        p = lax.axis_index("i")
        left = lax.rem(p + N_DEV - 1, N_DEV)
        right = lax.rem(p + 1, N_DEV)

        barrier_sem = pltpu.get_barrier_semaphore()
        for nbr in (left, right):
            pl.semaphore_signal(
                barrier_sem, inc=1,
                device_id=(nbr,), device_id_type=pl.DeviceIdType.MESH,
            )
        pl.semaphore_wait(barrier_sem, 2)

        bbf_ref[...] = b_ref[...].astype(jnp.bfloat16)

        def chunk(c):
            a = a_ref[pl.ds(c * ch, ch), :].astype(jnp.bfloat16)
            return jnp.dot(a, bbf_ref[...], preferred_element_type=jnp.float32)

        comm_ref[0, :, :] = chunk(lax.rem(p + N_DEV - 1, N_DEV)).astype(
            jnp.bfloat16
        )
        for s in range(N_DEV - 1):
            rdma = pltpu.make_async_remote_copy(
                src_ref=comm_ref.at[s],
                dst_ref=comm_ref.at[s + 1],
                send_sem=send_sems.at[s],
                recv_sem=recv_sems.at[s],
                device_id=(right,),
                device_id_type=pl.DeviceIdType.MESH,
            )
            rdma.start()
            c_in = lax.rem(p + 2 * N_DEV - 2 - s, N_DEV)
            local = chunk(c_in)
            rdma.wait()
            acc = comm_ref[s + 1, :, :].astype(jnp.float32) + local
            if s < N_DEV - 2:
                comm_ref[s + 1, :, :] = acc.astype(jnp.bfloat16)
            else:
                relu = jnp.maximum(acc, 0.0)
                out_ref[pl.ds(p * ch, ch), :] = relu
                comm_ref[s + 1, :, :] = relu.astype(jnp.bfloat16)

        for t in range(N_DEV - 1):
            rdma = pltpu.make_async_remote_copy(
                src_ref=comm_ref.at[3 + t],
                dst_ref=comm_ref.at[4 + t],
                send_sem=send_sems.at[3 + t],
                recv_sem=recv_sems.at[3 + t],
                device_id=(right,),
                device_id_type=pl.DeviceIdType.MESH,
            )
            rdma.start()
            rdma.wait()
            origin = lax.rem(p + 2 * N_DEV - 1 - t, N_DEV)
            out_ref[pl.ds(origin * ch, ch), :] = comm_ref[
                4 + t, :, :
            ].astype(jnp.float32)

    return pl.pallas_call(
        body,
        out_shape=jax.ShapeDtypeStruct((m, n), jnp.float32),
        in_specs=[
            pl.BlockSpec(memory_space=pltpu.VMEM),
            pl.BlockSpec(memory_space=pltpu.VMEM),
        ],
        out_specs=pl.BlockSpec(memory_space=pltpu.VMEM),
        scratch_shapes=[
            pltpu.VMEM((k, n), jnp.bfloat16),
            pltpu.VMEM((n_steps + 1, ch, n), jnp.bfloat16),
            pltpu.SemaphoreType.DMA((n_steps,)),
            pltpu.SemaphoreType.DMA((n_steps,)),
        ],
        compiler_params=pltpu.CompilerParams(collective_id=0),
    )(A, B)


# device time: 59615 ns/iter; 1.6758x vs baseline; 1.6758x over previous
import jax
import jax.numpy as jnp
from jax import lax
from jax.experimental import pallas as pl
from jax.experimental.pallas import tpu as pltpu

N_DEV = 4


def kernel(A, B):
    m, k = A.shape
    _, n = B.shape
    mh = m // 2
    mq = m // 4
    nh = n // 2

    def body(
        a_ref, b_ref, out_ref, bbf_ref,
        s1_send, s1_recv, acc_ref, s2_send, s2_recv,
        ag1_send, ag1_recv, ag2_send, ag2_recv,
        send_sems, recv_sems,
    ):
        p = lax.axis_index("i")
        nbr_a = jnp.bitwise_xor(p, 1)
        nbr_b = 3 - p

        barrier_sem = pltpu.get_barrier_semaphore()
        for nbr in (nbr_a, nbr_b):
            pl.semaphore_signal(
                barrier_sem, inc=1,
                device_id=(nbr,), device_id_type=pl.DeviceIdType.MESH,
            )
        pl.semaphore_wait(barrier_sem, 2)

        bbf_ref[...] = b_ref[...].astype(jnp.bfloat16)

        def params(b):
            if b == 0:
                p1, p2 = nbr_a, nbr_b
                half_lo = jnp.logical_or(p == 0, p == 3)
                q_lo = p < 2
            else:
                p1, p2 = nbr_b, nbr_a
                half_lo = p < 2
                q_lo = lax.rem(p, 2) == 0
            half_start = jnp.where(half_lo, 0, mh)
            rel_q = jnp.where(q_lo, 0, mq)
            return p1, p2, half_start, rel_q

        P = [params(0), params(1)]

        def mm(row_start, b):
            a = a_ref[pl.ds(row_start, mh), :].astype(jnp.bfloat16)
            return jnp.dot(
                a, bbf_ref[:, b * nh:(b + 1) * nh],
                preferred_element_type=jnp.float32,
            )

        def copy(src, dst, sem_idx, dev):
            return pltpu.make_async_remote_copy(
                src_ref=src, dst_ref=dst,
                send_sem=send_sems.at[sem_idx],
                recv_sem=recv_sems.at[sem_idx],
                device_id=(dev,), device_id_type=pl.DeviceIdType.MESH,
            )

        rs1 = []
        for b in range(2):
            p1, p2, half_start, rel_q = P[b]
            s1_send[b, :, :] = mm(mh - half_start, b).astype(jnp.bfloat16)
            r = copy(s1_send.at[b], s1_recv.at[b], b, p1)
            r.start()
            rs1.append(r)

        for b in range(2):
            _, _, half_start, _ = P[b]
            acc_ref[b, :, :] = mm(half_start, b)

        rs2 = []
        for b in range(2):
            p1, p2, half_start, rel_q = P[b]
            rs1[b].wait_recv()
            acc_ref[b, :, :] = (
                acc_ref[b, :, :] + s1_recv[b, :, :].astype(jnp.float32)
            )
            s2_send[b, :, :] = acc_ref[
                b, pl.ds(mq - rel_q, mq), :
            ].astype(jnp.bfloat16)
            r = copy(s2_send.at[b], s2_recv.at[b], 2 + b, p2)
            r.start()
            rs2.append(r)

        ag1 = []
        for b in range(2):
            p1, p2, half_start, rel_q = P[b]
            rs2[b].wait_recv()
            q_sum = (
                acc_ref[b, pl.ds(rel_q, mq), :]
                + s2_recv[b, :, :].astype(jnp.float32)
            )
            relu = jnp.maximum(q_sum, 0.0)
            out_ref[pl.ds(half_start + rel_q, mq), b * nh:(b + 1) * nh] = relu
            ag1_send[b, :, :] = relu.astype(jnp.bfloat16)
            r = copy(ag1_send.at[b], ag1_recv.at[b], 4 + b, p2)
            r.start()
            ag1.append(r)

        ag2 = []
        for b in range(2):
            p1, p2, half_start, rel_q = P[b]
            ag1[b].wait_recv()
            out_ref[
                pl.ds(half_start + mq - rel_q, mq), b * nh:(b + 1) * nh
            ] = ag1_recv[b, :, :].astype(jnp.float32)
            ag2_send[b, pl.ds(rel_q, mq), :] = ag1_send[b, :, :]
            ag2_send[b, pl.ds(mq - rel_q, mq), :] = ag1_recv[b, :, :]
            r = copy(ag2_send.at[b], ag2_recv.at[b], 6 + b, p1)
            r.start()
            ag2.append(r)

        for b in range(2):
            p1, p2, half_start, rel_q = P[b]
            ag2[b].wait_recv()
            out_ref[
                pl.ds(mh - half_start, mh), b * nh:(b + 1) * nh
            ] = ag2_recv[b, :, :].astype(jnp.float32)

        for r in rs1 + rs2 + ag1 + ag2:
            r.wait_send()

    return pl.pallas_call(
        body,
        out_shape=jax.ShapeDtypeStruct((m, n), jnp.float32),
        in_specs=[
            pl.BlockSpec(memory_space=pltpu.VMEM),
            pl.BlockSpec(memory_space=pltpu.VMEM),
        ],
        out_specs=pl.BlockSpec(memory_space=pltpu.VMEM),
        scratch_shapes=[
            pltpu.VMEM((k, n), jnp.bfloat16),
            pltpu.VMEM((2, mh, nh), jnp.bfloat16),
            pltpu.VMEM((2, mh, nh), jnp.bfloat16),
            pltpu.VMEM((2, mh, nh), jnp.float32),
            pltpu.VMEM((2, mq, nh), jnp.bfloat16),
            pltpu.VMEM((2, mq, nh), jnp.bfloat16),
            pltpu.VMEM((2, mq, nh), jnp.bfloat16),
            pltpu.VMEM((2, mq, nh), jnp.bfloat16),
            pltpu.VMEM((2, mh, nh), jnp.bfloat16),
            pltpu.VMEM((2, mh, nh), jnp.bfloat16),
            pltpu.SemaphoreType.DMA((8,)),
            pltpu.SemaphoreType.DMA((8,)),
        ],
        compiler_params=pltpu.CompilerParams(collective_id=0),
    )(A, B)


# device time: 57822 ns/iter; 1.7278x vs baseline; 1.0310x over previous
import jax
import jax.numpy as jnp
from jax import lax
from jax.experimental import pallas as pl
from jax.experimental.pallas import tpu as pltpu

N_DEV = 4


def kernel(A, B):
    m, k = A.shape
    _, n = B.shape
    mh = m // 2
    mq = m // 4
    nh = n // 2

    def body(
        a_ref, b_ref, out_ref, bbf_ref,
        s1_send, s1_recv, acc_ref, s2_send, s2_recv,
        send_sems, recv_sems,
    ):
        p = lax.axis_index("i")
        nbr_a = jnp.bitwise_xor(p, 1)
        nbr_b = 3 - p

        barrier_sem = pltpu.get_barrier_semaphore()
        for nbr in (nbr_a, nbr_b):
            pl.semaphore_signal(
                barrier_sem, inc=1,
                device_id=(nbr,), device_id_type=pl.DeviceIdType.MESH,
            )
        pl.semaphore_wait(barrier_sem, 2)

        bbf_ref[...] = b_ref[...].astype(jnp.bfloat16)

        def params(b):
            if b == 0:
                p1, p2 = nbr_a, nbr_b
                half_lo = jnp.logical_or(p == 0, p == 3)
                q_lo = p < 2
            else:
                p1, p2 = nbr_b, nbr_a
                half_lo = p < 2
                q_lo = lax.rem(p, 2) == 0
            half_start = jnp.where(half_lo, 0, mh)
            rel_q = jnp.where(q_lo, 0, mq)
            return p1, p2, half_start, rel_q

        P = [params(0), params(1)]

        def mm(row_start, b):
            a = a_ref[pl.ds(row_start, mh), :].astype(jnp.bfloat16)
            return jnp.dot(
                a, bbf_ref[:, b * nh:(b + 1) * nh],
                preferred_element_type=jnp.float32,
            )

        def copy(src, dst, sem_idx, dev):
            return pltpu.make_async_remote_copy(
                src_ref=src, dst_ref=dst,
                send_sem=send_sems.at[sem_idx],
                recv_sem=recv_sems.at[sem_idx],
                device_id=(dev,), device_id_type=pl.DeviceIdType.MESH,
            )

        rs1 = []
        for b in range(2):
            p1, p2, half_start, rel_q = P[b]
            s1_send[b, :, :] = mm(mh - half_start, b).astype(jnp.bfloat16)
            r = copy(s1_send.at[b], s1_recv.at[b], b, p1)
            r.start()
            rs1.append(r)

        for b in range(2):
            _, _, half_start, _ = P[b]
            acc_ref[b, :, :] = mm(half_start, b)

        rs2 = []
        for b in range(2):
            p1, p2, half_start, rel_q = P[b]
            rs1[b].wait_recv()
            acc_ref[b, :, :] = (
                acc_ref[b, :, :] + s1_recv[b, :, :].astype(jnp.float32)
            )
            s2_send[b, :, :] = acc_ref[
                b, pl.ds(mq - rel_q, mq), :
            ].astype(jnp.bfloat16)
            r = copy(s2_send.at[b], s2_recv.at[b], 2 + b, p2)
            r.start()
            rs2.append(r)

        ag1 = []
        for b in range(2):
            p1, p2, half_start, rel_q = P[b]
            rs2[b].wait_recv()
            q_sum = (
                acc_ref[b, pl.ds(rel_q, mq), :]
                + s2_recv[b, :, :].astype(jnp.float32)
            )
            q_start = half_start + rel_q
            out_ref[pl.ds(q_start, mq), b * nh:(b + 1) * nh] = jnp.maximum(
                q_sum, 0.0
            ).astype(jnp.bfloat16)
            q_slice = out_ref.at[pl.ds(q_start, mq), b * nh:(b + 1) * nh]
            r = copy(q_slice, q_slice, 4 + b, p2)
            r.start()
            ag1.append(r)

        ag2 = []
        for b in range(2):
            p1, p2, half_start, rel_q = P[b]
            ag1[b].wait_recv()
            h_slice = out_ref.at[pl.ds(half_start, mh), b * nh:(b + 1) * nh]
            r = copy(h_slice, h_slice, 6 + b, p1)
            r.start()
            ag2.append(r)

        for b in range(2):
            ag2[b].wait_recv()

        for r in rs1 + rs2 + ag1 + ag2:
            r.wait_send()

    return pl.pallas_call(
        body,
        out_shape=jax.ShapeDtypeStruct((m, n), jnp.bfloat16),
        in_specs=[
            pl.BlockSpec(memory_space=pltpu.VMEM),
            pl.BlockSpec(memory_space=pltpu.VMEM),
        ],
        out_specs=pl.BlockSpec(memory_space=pltpu.VMEM),
        scratch_shapes=[
            pltpu.VMEM((k, n), jnp.bfloat16),
            pltpu.VMEM((2, mh, nh), jnp.bfloat16),
            pltpu.VMEM((2, mh, nh), jnp.bfloat16),
            pltpu.VMEM((2, mh, nh), jnp.float32),
            pltpu.VMEM((2, mq, nh), jnp.bfloat16),
            pltpu.VMEM((2, mq, nh), jnp.bfloat16),
            pltpu.SemaphoreType.DMA((8,)),
            pltpu.SemaphoreType.DMA((8,)),
        ],
        compiler_params=pltpu.CompilerParams(collective_id=0),
    )(A, B)


# device time: 51819 ns/iter; 1.9280x vs baseline; 1.1158x over previous
import contextlib
import os

import jax
import jax.numpy as jnp
from jax import lax
from jax.experimental import pallas as pl
from jax.experimental.pallas import tpu as pltpu

N_DEV = 4
_PROFILE = os.environ.get("PROFILE_SCOPES") == "1"


def _scope(name):
    return jax.named_scope(name) if _PROFILE else contextlib.nullcontext()


def kernel(A, B):
    m, k = A.shape
    _, n = B.shape
    mh = m // 2
    mq = m // 4
    nh = n // 2
    nq = nh // 2

    ORDER = [(0, 0), (1, 0), (0, 1), (1, 1)]

    def body(
        a_ref, b_ref, out_ref, bbf_ref,
        s1_send, s1_recv, acc_ref, s2_send, s2_recv,
        send_sems, recv_sems,
    ):
        p = lax.axis_index("i")
        nbr_a = jnp.bitwise_xor(p, 1)
        nbr_b = 3 - p

        with _scope("barrier"):
            barrier_sem = pltpu.get_barrier_semaphore()
            for nbr in (nbr_a, nbr_b):
                pl.semaphore_signal(
                    barrier_sem, inc=1,
                    device_id=(nbr,), device_id_type=pl.DeviceIdType.MESH,
                )
            pl.semaphore_wait(barrier_sem, 2)

        def params(b):
            if b == 0:
                p1, p2 = nbr_a, nbr_b
                half_lo = jnp.logical_or(p == 0, p == 3)
                q_lo = p < 2
            else:
                p1, p2 = nbr_b, nbr_a
                half_lo = p < 2
                q_lo = lax.rem(p, 2) == 0
            half_start = jnp.where(half_lo, 0, mh)
            rel_q = jnp.where(q_lo, 0, mq)
            return p1, p2, half_start, rel_q

        P = [params(0), params(1)]

        def col0(b, c):
            return b * nh + c * nq

        def mm(row_start, b, c):
            a = a_ref[pl.ds(row_start, mh), :].astype(jnp.bfloat16)
            return jnp.dot(
                a, bbf_ref[:, col0(b, c):col0(b, c) + nq],
                preferred_element_type=jnp.float32,
            )

        def copy(src, dst, sem_idx, dev):
            return pltpu.make_async_remote_copy(
                src_ref=src, dst_ref=dst,
                send_sem=send_sems.at[sem_idx],
                recv_sem=recv_sems.at[sem_idx],
                device_id=(dev,), device_id_type=pl.DeviceIdType.MESH,
            )

        rs1 = {}
        for i, (b, c) in enumerate(ORDER):
            p1, p2, half_start, rel_q = P[b]
            with _scope(f"mm_send#i={i}"):
                bbf_ref[:, col0(b, c):col0(b, c) + nq] = b_ref[
                    :, col0(b, c):col0(b, c) + nq
                ].astype(jnp.bfloat16)
                s1_send[b, c] = mm(mh - half_start, b, c).astype(jnp.bfloat16)
            r = copy(s1_send.at[b, c], s1_recv.at[b, c], i, p1)
            r.start()
            rs1[b, c] = r

        for i, (b, c) in enumerate(ORDER):
            _, _, half_start, _ = P[b]
            with _scope(f"mm_keep#i={i}"):
                acc_ref[b, c] = mm(half_start, b, c)

        rs2 = {}
        for i, (b, c) in enumerate(ORDER):
            p1, p2, half_start, rel_q = P[b]
            with _scope(f"wait_rs1#i={i}"):
                rs1[b, c].wait_recv()
            with _scope(f"add1#i={i}"):
                sq = pl.ds(mq - rel_q, mq)
                s2_send[b, c] = (
                    acc_ref[b, c, sq, :]
                    + s1_recv[b, c, sq, :].astype(jnp.float32)
                ).astype(jnp.bfloat16)
            r = copy(s2_send.at[b, c], s2_recv.at[b, c], 4 + i, p2)
            r.start()
            rs2[b, c] = r

        ag1 = {}
        for i, (b, c) in enumerate(ORDER):
            p1, p2, half_start, rel_q = P[b]
            with _scope(f"wait_rs2#i={i}"):
                rs2[b, c].wait_recv()
            with _scope(f"relu#i={i}"):
                kq = pl.ds(rel_q, mq)
                q_sum = (
                    acc_ref[b, c, kq, :]
                    + s1_recv[b, c, kq, :].astype(jnp.float32)
                    + s2_recv[b, c].astype(jnp.float32)
                )
                q_start = half_start + rel_q
                out_ref[
                    pl.ds(q_start, mq), col0(b, c):col0(b, c) + nq
                ] = jnp.maximum(q_sum, 0.0).astype(jnp.bfloat16)
            q_slice = out_ref.at[pl.ds(q_start, mq), col0(b, c):col0(b, c) + nq]
            r = copy(q_slice, q_slice, 8 + i, p2)
            r.start()
            ag1[b, c] = r

        ag2 = {}
        for i, (b, c) in enumerate(ORDER):
            p1, p2, half_start, rel_q = P[b]
            with _scope(f"wait_ag1#i={i}"):
                ag1[b, c].wait_recv()
            h_slice = out_ref.at[
                pl.ds(half_start, mh), col0(b, c):col0(b, c) + nq
            ]
            r = copy(h_slice, h_slice, 12 + i, p1)
            r.start()
            ag2[b, c] = r

        for i, (b, c) in enumerate(ORDER):
            with _scope(f"wait_ag2#i={i}"):
                ag2[b, c].wait_recv()

        with _scope("drain"):
            for r in (
                list(rs1.values()) + list(rs2.values())
                + list(ag1.values()) + list(ag2.values())
            ):
                r.wait_send()

    return pl.pallas_call(
        body,
        out_shape=jax.ShapeDtypeStruct((m, n), jnp.bfloat16),
        in_specs=[
            pl.BlockSpec(memory_space=pltpu.VMEM),
            pl.BlockSpec(memory_space=pltpu.VMEM),
        ],
        out_specs=pl.BlockSpec(memory_space=pltpu.VMEM),
        scratch_shapes=[
            pltpu.VMEM((k, n), jnp.bfloat16),
            pltpu.VMEM((2, 2, mh, nq), jnp.bfloat16),
            pltpu.VMEM((2, 2, mh, nq), jnp.bfloat16),
            pltpu.VMEM((2, 2, mh, nq), jnp.float32),
            pltpu.VMEM((2, 2, mq, nq), jnp.bfloat16),
            pltpu.VMEM((2, 2, mq, nq), jnp.bfloat16),
            pltpu.SemaphoreType.DMA((16,)),
            pltpu.SemaphoreType.DMA((16,)),
        ],
        compiler_params=pltpu.CompilerParams(collective_id=0),
    )(A, B)
